# baseline (speedup 1.0000x reference)
import functools

import jax
import jax.numpy as jnp
from jax.experimental import pallas as pl
from jax.experimental.pallas import tpu as pltpu

_R = 1
_BR = 200


def _graph_conv_kernel(*refs):
    i = pl.program_id(0)
    a_refs = refs[:2 * _R]
    xf_ref, w0t_ref, w1t_ref, b_ref, o_ref = refs[2 * _R:]
    xf32_ref = xf_ref
    w0t = w0t_ref[:]
    w1t = w1t_ref[:]
    bias = b_ref[:]
    bn = _R * _BR
    for r in range(_R):
        p0 = jax.lax.dot_general(
            a_refs[r][0], xf32_ref[:], (((1,), (0,)), ((), ())),
            precision=jax.lax.Precision.DEFAULT,
            preferred_element_type=jnp.float32)
        p1 = jax.lax.dot_general(
            a_refs[_R + r][0], xf32_ref[:], (((1,), (0,)), ((), ())),
            precision=jax.lax.Precision.DEFAULT,
            preferred_element_type=jnp.float32)
        xi = xf_ref[pl.ds(i * bn + r * _BR, _BR), :]
        o_ref[r * _BR:(r + 1) * _BR, :] = (
            jnp.dot(p0 + xi, w0t, preferred_element_type=jnp.float32)
            + jnp.dot(p1 + xi, w1t, preferred_element_type=jnp.float32)
            + bias
        )


@jax.jit
def _graph_conv(x0, supports, w0t, w1t, b2d):
    n, d = x0.shape
    out = w0t.shape[1]
    bn = _R * _BR
    a_specs = [
        pl.BlockSpec((1, _BR, n), functools.partial(
            lambda i, s=0, r=0: (s, i * _R + r, 0), s=s, r=r))
        for s in range(2)
        for r in range(_R)
    ]
    return pl.pallas_call(
        _graph_conv_kernel,
        grid=(pl.cdiv(n, bn),),
        in_specs=a_specs + [
            pl.BlockSpec((n, d), lambda i: (0, 0)),
            pl.BlockSpec((d, out), lambda i: (0, 0)),
            pl.BlockSpec((d, out), lambda i: (0, 0)),
            pl.BlockSpec((1, out), lambda i: (0, 0)),
        ],
        out_specs=pl.BlockSpec((bn, out), lambda i: (i, 0)),
        out_shape=jax.ShapeDtypeStruct((n, out), jnp.float32),
        compiler_params=pltpu.CompilerParams(
            dimension_semantics=("arbitrary",),
        ),
    )(*([supports] * (2 * _R)), x0, w0t, w1t, b2d)


def kernel(inputs, supports, W, b):
    bsz, n, d = inputs.shape
    s = supports.shape[0]
    out_dim = W.shape[0]
    if bsz == 1:
        x0 = inputs.reshape(n, d)
    else:
        x0 = jnp.transpose(inputs, (1, 2, 0)).reshape(n, d * bsz)
    w0t = jnp.transpose(W[:, 0::s])
    w1t = jnp.transpose(W[:, 1::s])
    b2d = b.reshape(1, out_dim)

    res = _graph_conv(x0, supports, w0t, w1t, b2d)
    return res.reshape(bsz, n, out_dim)


# speedup vs baseline: 1.0062x; 1.0062x over previous
import functools

import jax
import jax.numpy as jnp
from jax.experimental import pallas as pl
from jax.experimental.pallas import tpu as pltpu

_R = 1
_BR = 200


def _graph_conv_kernel(*refs):
    i = pl.program_id(0)
    a_refs = refs[:2 * _R]
    xf_ref, w0t_ref, w1t_ref, b_ref, o_ref = refs[2 * _R:]
    w0t = w0t_ref[:]
    w1t = w1t_ref[:]
    bias = b_ref[:]
    bn = _R * _BR
    for r in range(_R):
        p0 = jax.lax.dot_general(
            a_refs[r][0], xf_ref[:], (((1,), (0,)), ((), ())),
            precision=jax.lax.Precision.DEFAULT,
            preferred_element_type=jnp.float32)
        p1 = jax.lax.dot_general(
            a_refs[_R + r][0], xf_ref[:], (((1,), (0,)), ((), ())),
            precision=jax.lax.Precision.DEFAULT,
            preferred_element_type=jnp.float32)
        xi = xf_ref[pl.ds(i * bn + r * _BR, _BR), :]
        o_ref[r * _BR:(r + 1) * _BR, :] = (
            jnp.dot(p0 + xi, w0t, preferred_element_type=jnp.float32)
            + jnp.dot(p1 + xi, w1t, preferred_element_type=jnp.float32)
            + bias
        )


@jax.jit
def _graph_conv(x0, supports, w0t, w1t, b2d):
    n, d = x0.shape
    out = w0t.shape[1]
    bn = _R * _BR
    a_specs = [
        pl.BlockSpec((1, _BR, n), functools.partial(
            lambda i, s=0, r=0: (s, i * _R + r, 0), s=s, r=r))
        for s in range(2)
        for r in range(_R)
    ]
    return pl.pallas_call(
        _graph_conv_kernel,
        grid=(pl.cdiv(n, bn),),
        in_specs=a_specs + [
            pl.BlockSpec((n, d), lambda i: (0, 0)),
            pl.BlockSpec((d, out), lambda i: (0, 0)),
            pl.BlockSpec((d, out), lambda i: (0, 0)),
            pl.BlockSpec((1, out), lambda i: (0, 0)),
        ],
        out_specs=pl.BlockSpec((bn, out), lambda i: (i, 0)),
        out_shape=jax.ShapeDtypeStruct((n, out), jnp.float32),
        compiler_params=pltpu.CompilerParams(
            dimension_semantics=("arbitrary",),
        ),
    )(*([supports] * (2 * _R)), x0, w0t, w1t, b2d)


def kernel(inputs, supports, W, b):
    bsz, n, d = inputs.shape
    s = supports.shape[0]
    out_dim = W.shape[0]
    if bsz == 1:
        x0 = inputs.reshape(n, d)
    else:
        x0 = jnp.transpose(inputs, (1, 2, 0)).reshape(n, d * bsz)
    w0t = jnp.transpose(W[:, 0::s])
    w1t = jnp.transpose(W[:, 1::s])
    b2d = b.reshape(1, out_dim)

    res = _graph_conv(x0, supports, w0t, w1t, b2d)
    return res.reshape(bsz, n, out_dim)
